# Initial kernel scaffold; baseline (speedup 1.0000x reference)
import jax
import jax.numpy as jnp
from jax.experimental import pallas as pl


def kernel(feat0, feat1, mp0_row, mp0_col, mp0_val, mp1_row, mp1_col, mp1_val, mask_idx0, mask_idx1, mask_token0, mask_token1, meta_emb0, meta_emb1, W0, W1):
    raise NotImplementedError("write your pallas kernel here")


# speedup vs baseline: 3.7148x; 3.7148x over previous
import functools

import jax
import jax.numpy as jnp
from jax import lax
from jax.experimental import pallas as pl
from jax.experimental.pallas import tpu as pltpu
from jax.experimental.pallas import tpu_sc as plsc

STRENGTH = 0.1

NC = 2
NS = 16
LANES = 16

EDGE_WIN = 80


def _mask_project_body(feat_ref, midx_ref, tok_ref, w_ref, masked_ref, proj_ref,
                       *, block_rows):
    b = pl.program_id(0)
    feat = feat_ref[...]
    midx = midx_ref[0, :]
    rows = b * block_rows + lax.broadcasted_iota(
        jnp.int32, (block_rows, midx.shape[0]), 0)
    is_masked = jnp.any(rows == midx[None, :], axis=1)
    tok = tok_ref[0, :]
    masked = jnp.where(is_masked[:, None], tok[None, :], feat)
    masked_ref[...] = masked
    proj_ref[...] = lax.dot_general(
        masked, w_ref[...], (((1,), (1,)), ((), ())),
        preferred_element_type=jnp.float32,
        precision=lax.Precision.HIGHEST)


def _mask_project(feat, mask_idx, mask_token, w, block_rows=1000):
    n, d = feat.shape
    nb = n // block_rows
    nmask = mask_idx.shape[0]
    return pl.pallas_call(
        functools.partial(_mask_project_body, block_rows=block_rows),
        grid=(nb,),
        in_specs=[
            pl.BlockSpec((block_rows, d), lambda b: (b, 0)),
            pl.BlockSpec((1, nmask), lambda b: (0, 0)),
            pl.BlockSpec((1, d), lambda b: (0, 0)),
            pl.BlockSpec((d, d), lambda b: (0, 0)),
        ],
        out_specs=[
            pl.BlockSpec((block_rows, d), lambda b: (b, 0)),
            pl.BlockSpec((block_rows, d), lambda b: (b, 0)),
        ],
        out_shape=[
            jax.ShapeDtypeStruct((n, d), jnp.float32),
            jax.ShapeDtypeStruct((n, d), jnp.float32),
        ],
    )(feat, mask_idx.reshape(1, nmask), mask_token, w)


def _spmm_body(proj0, proj1, row0, col0, val0, row1, col1, val1,
               part0, part1,
               ridx, cidx, vbuf, rows_v, zbuf, acc, sem,
               *, n_pad, d, e):
    c = lax.axis_index("c")
    s = lax.axis_index("s")
    wid = c * NS + s
    e_per_w = e // (NC * NS)
    n_win = e_per_w // EDGE_WIN
    rows_per_sub = n_pad // NS
    zrows = zbuf.shape[0]

    def zfill(r, _):
        for j in range(d // LANES):
            zbuf[r, pl.ds(j * LANES, LANES)] = jnp.zeros((LANES,), jnp.float32)
        return 0
    lax.fori_loop(0, zrows, zfill, 0)

    def run_path(proj, row, col, val, part, acc):
        def zero_step(t, _):
            pltpu.sync_copy(zbuf, acc.at[pl.ds(s * rows_per_sub + t * zrows, zrows)])
            return 0
        lax.fori_loop(0, rows_per_sub // zrows, zero_step, 0)
        plsc.subcore_barrier()

        def win(i, _):
            base = wid * e_per_w + i * EDGE_WIN
            pltpu.sync_copy(col.at[pl.ds(base, EDGE_WIN)], cidx)
            pltpu.sync_copy(row.at[pl.ds(base, EDGE_WIN)], ridx)
            pltpu.sync_copy(val.at[pl.ds(base, EDGE_WIN)], vbuf)
            pltpu.async_copy(proj.at[cidx], rows_v, sem).wait()

            def scale(g, _):
                vb = vbuf[pl.ds(g * LANES, LANES)]
                for t in range(LANES):
                    k = g * LANES + t
                    v = jnp.full((LANES,), vb[t], jnp.float32)
                    for j in range(d // LANES):
                        rows_v[k, pl.ds(j * LANES, LANES)] = (
                            rows_v[k, pl.ds(j * LANES, LANES)] * v)
                return 0
            lax.fori_loop(0, EDGE_WIN // LANES, scale, 0)

            pltpu.sync_copy(rows_v, acc.at[ridx], add=True)
            return 0
        lax.fori_loop(0, n_win, win, 0)
        plsc.subcore_barrier()

        pltpu.sync_copy(acc.at[pl.ds(s * rows_per_sub, rows_per_sub)],
                        part.at[c, pl.ds(s * rows_per_sub, rows_per_sub)])
        plsc.subcore_barrier()

    run_path(proj0, row0, col0, val0, part0, acc)
    run_path(proj1, row1, col1, val1, part1, acc)


def _spmm_both(proj0, proj1, row0, col0, val0, row1, col1, val1):
    n, d = proj0.shape
    e = row0.shape[0]
    n_pad = ((n + 8 * NS - 1) // (8 * NS)) * (8 * NS)
    mesh = plsc.VectorSubcoreMesh(core_axis_name="c", subcore_axis_name="s",
                                  num_cores=NC, num_subcores=NS)
    kern = pl.kernel(
        functools.partial(_spmm_body, n_pad=n_pad, d=d, e=e),
        out_type=[
            jax.ShapeDtypeStruct((NC, n_pad, d), jnp.float32),
            jax.ShapeDtypeStruct((NC, n_pad, d), jnp.float32),
        ],
        mesh=mesh,
        scratch_types=[
            pltpu.VMEM((EDGE_WIN,), jnp.int32),
            pltpu.VMEM((EDGE_WIN,), jnp.int32),
            pltpu.VMEM((EDGE_WIN,), jnp.float32),
            pltpu.VMEM((EDGE_WIN, d), jnp.float32),
            pltpu.VMEM((8, d), jnp.float32),
            pltpu.VMEM_SHARED((n_pad, d), jnp.float32),
            pltpu.SemaphoreType.DMA,
        ],
    )
    return kern(proj0, proj1, row0, col0, val0, row1, col1, val1)


def _combine_body(masked_ref, part_ref, meta_ref, out_ref):
    meta = meta_ref[0, :]
    out_ref[...] = masked_ref[...] + STRENGTH * (
        part_ref[0] + part_ref[1] + meta[None, :])


def _combine(masked, part, meta_emb, block_rows=1000):
    n, d = masked.shape
    nb = n // block_rows
    return pl.pallas_call(
        _combine_body,
        grid=(nb,),
        in_specs=[
            pl.BlockSpec((block_rows, d), lambda b: (b, 0)),
            pl.BlockSpec((NC, block_rows, d), lambda b: (0, b, 0)),
            pl.BlockSpec((1, d), lambda b: (0, 0)),
        ],
        out_specs=pl.BlockSpec((block_rows, d), lambda b: (b, 0)),
        out_shape=jax.ShapeDtypeStruct((n, d), jnp.float32),
    )(masked, part, meta_emb)


def kernel(feat0, feat1, mp0_row, mp0_col, mp0_val, mp1_row, mp1_col, mp1_val,
           mask_idx0, mask_idx1, mask_token0, mask_token1,
           meta_emb0, meta_emb1, W0, W1):
    masked0, proj0 = _mask_project(feat0, mask_idx0, mask_token0, W0)
    masked1, proj1 = _mask_project(feat1, mask_idx1, mask_token1, W1)
    part0, part1 = _spmm_both(proj0, proj1, mp0_row, mp0_col, mp0_val,
                              mp1_row, mp1_col, mp1_val)
    out0 = _combine(masked0, part0, meta_emb0)
    out1 = _combine(masked1, part1, meta_emb1)
    return (out0, out1)


# speedup vs baseline: 9.0948x; 2.4482x over previous
import functools

import jax
import jax.numpy as jnp
from jax import lax
from jax.experimental import pallas as pl
from jax.experimental.pallas import tpu as pltpu
from jax.experimental.pallas import tpu_sc as plsc

STRENGTH = 0.1

NC = 2
NS = 16
LANES = 16

EDGE_WIN = 80


def _mask_project_body(feat_ref, midx_ref, tok_ref, w_ref, masked_ref, proj_ref,
                       *, block_rows):
    b = pl.program_id(0)
    feat = feat_ref[...]
    midx = midx_ref[0, :]
    rows = b * block_rows + lax.broadcasted_iota(
        jnp.int32, (block_rows, midx.shape[0]), 0)
    is_masked = jnp.any(rows == midx[None, :], axis=1)
    tok = tok_ref[0, :]
    masked = jnp.where(is_masked[:, None], tok[None, :], feat)
    masked_ref[...] = masked
    proj_ref[...] = lax.dot_general(
        masked, w_ref[...], (((1,), (1,)), ((), ())),
        preferred_element_type=jnp.float32,
        precision=lax.Precision.HIGHEST)


def _mask_project(feat, mask_idx, mask_token, w, block_rows=1000):
    n, d = feat.shape
    nb = n // block_rows
    nmask = mask_idx.shape[0]
    return pl.pallas_call(
        functools.partial(_mask_project_body, block_rows=block_rows),
        grid=(nb,),
        in_specs=[
            pl.BlockSpec((block_rows, d), lambda b: (b, 0)),
            pl.BlockSpec((1, nmask), lambda b: (0, 0)),
            pl.BlockSpec((1, d), lambda b: (0, 0)),
            pl.BlockSpec((d, d), lambda b: (0, 0)),
        ],
        out_specs=[
            pl.BlockSpec((block_rows, d), lambda b: (b, 0)),
            pl.BlockSpec((block_rows, d), lambda b: (b, 0)),
        ],
        out_shape=[
            jax.ShapeDtypeStruct((n, d), jnp.float32),
            jax.ShapeDtypeStruct((n, d), jnp.float32),
        ],
    )(feat, mask_idx.reshape(1, nmask), mask_token, w)


def _spmm_body(proj0, proj1, row0, col0, val0, row1, col1, val1,
               part0, part1,
               ridx0, ridx1, cidx0, cidx1, vbuf0, vbuf1, sidx0, sidx1,
               rows0, rows1, zbuf, acc,
               semi0, semi1, semg0, semg1, sems0, sems1,
               *, n_pad, d, e):
    c = lax.axis_index("c")
    s = lax.axis_index("s")
    wid = c * NS + s
    e_per_w = e // (NC * NS)
    n_win = e_per_w // EDGE_WIN
    rows_per_sub = n_pad // NS
    zrows = zbuf.shape[0]
    n_grp = EDGE_WIN // LANES

    ridx = (ridx0, ridx1)
    cidx = (cidx0, cidx1)
    vbuf = (vbuf0, vbuf1)
    sidx = (sidx0, sidx1)
    rows = (rows0, rows1)
    semi = (semi0, semi1)
    semg = (semg0, semg1)
    sems = (sems0, sems1)

    def zfill(r, _):
        for j in range(d // LANES):
            zbuf[r, pl.ds(j * LANES, LANES)] = jnp.zeros((LANES,), jnp.float32)
        return 0
    lax.fori_loop(0, zrows, zfill, 0)

    def run_path(proj, row, col, val, part):
        base0 = wid * e_per_w

        def fire_idx(w, b):
            off = base0 + w * EDGE_WIN
            pltpu.async_copy(row.at[pl.ds(off, EDGE_WIN)], ridx[b], semi[b])
            pltpu.async_copy(col.at[pl.ds(off, EDGE_WIN)], cidx[b], semi[b])
            pltpu.async_copy(val.at[pl.ds(off, EDGE_WIN)], vbuf[b], semi[b])

        def wait_idx(w, b):
            off = base0 + w * EDGE_WIN
            pltpu.make_async_copy(row.at[pl.ds(off, EDGE_WIN)], ridx[b], semi[b]).wait()
            pltpu.make_async_copy(col.at[pl.ds(off, EDGE_WIN)], cidx[b], semi[b]).wait()
            pltpu.make_async_copy(val.at[pl.ds(off, EDGE_WIN)], vbuf[b], semi[b]).wait()

        def fire_gather(b):
            pltpu.async_copy(proj.at[cidx[b]], rows[b], semg[b])

        def wait_gather(b):
            pltpu.make_async_copy(proj.at[cidx[b]], rows[b], semg[b]).wait()

        def fire_scat(b):
            pltpu.async_copy(rows[b], acc.at[sidx[b]], sems[b], add=True)

        def wait_scat(b):
            pltpu.make_async_copy(rows[b], acc.at[sidx[b]], sems[b]).wait()

        def scale_and_stage(b):
            def grp(g, _):
                vv = vbuf[b][pl.ds(g * LANES, LANES)]
                sidx[b][pl.ds(g * LANES, LANES)] = ridx[b][pl.ds(g * LANES, LANES)]
                for t in range(LANES):
                    k = g * LANES + t
                    v = jnp.full((LANES,), vv[t], jnp.float32)
                    for j in range(d // LANES):
                        rows[b][k, pl.ds(j * LANES, LANES)] = (
                            rows[b][k, pl.ds(j * LANES, LANES)] * v)
                return 0
            lax.fori_loop(0, n_grp, grp, 0)

        def zero_step(t, _):
            pltpu.sync_copy(zbuf, acc.at[pl.ds(s * rows_per_sub + t * zrows, zrows)])
            return 0
        lax.fori_loop(0, rows_per_sub // zrows, zero_step, 0)
        plsc.subcore_barrier()

        fire_idx(0, 0)
        fire_idx(1, 1)
        wait_idx(0, 0)
        fire_gather(0)
        wait_idx(1, 1)
        fire_gather(1)
        wait_gather(0)
        scale_and_stage(0)
        fire_scat(0)
        fire_idx(2, 0)

        def pair(t, _):
            for (wofs, b) in ((1, 1), (2, 0)):
                w = 2 * t + wofs
                ob = 1 - b

                @pl.when(w + 1 < n_win)
                def _():
                    wait_idx(w + 1, ob)
                    wait_scat(ob)
                    fire_gather(ob)

                wait_gather(b)
                scale_and_stage(b)
                fire_scat(b)

                @pl.when(w + 2 < n_win)
                def _():
                    fire_idx(w + 2, b)
            return 0
        lax.fori_loop(0, (n_win - 1) // 2, pair, 0)

        wait_scat(1)
        wait_scat(0)
        plsc.subcore_barrier()

        pltpu.sync_copy(acc.at[pl.ds(s * rows_per_sub, rows_per_sub)],
                        part.at[c, pl.ds(s * rows_per_sub, rows_per_sub)])
        plsc.subcore_barrier()

    run_path(proj0, row0, col0, val0, part0)
    run_path(proj1, row1, col1, val1, part1)


def _spmm_both(proj0, proj1, row0, col0, val0, row1, col1, val1):
    n, d = proj0.shape
    e = row0.shape[0]
    n_pad = ((n + 8 * NS - 1) // (8 * NS)) * (8 * NS)
    e_per_w = e // (NC * NS)
    n_win = e_per_w // EDGE_WIN
    assert e_per_w % EDGE_WIN == 0 and n_win >= 3 and n_win % 2 == 1
    mesh = plsc.VectorSubcoreMesh(core_axis_name="c", subcore_axis_name="s",
                                  num_cores=NC, num_subcores=NS)
    kern = pl.kernel(
        functools.partial(_spmm_body, n_pad=n_pad, d=d, e=e),
        out_type=[
            jax.ShapeDtypeStruct((NC, n_pad, d), jnp.float32),
            jax.ShapeDtypeStruct((NC, n_pad, d), jnp.float32),
        ],
        mesh=mesh,
        scratch_types=[
            pltpu.VMEM((EDGE_WIN,), jnp.int32),
            pltpu.VMEM((EDGE_WIN,), jnp.int32),
            pltpu.VMEM((EDGE_WIN,), jnp.int32),
            pltpu.VMEM((EDGE_WIN,), jnp.int32),
            pltpu.VMEM((EDGE_WIN,), jnp.float32),
            pltpu.VMEM((EDGE_WIN,), jnp.float32),
            pltpu.VMEM((EDGE_WIN,), jnp.int32),
            pltpu.VMEM((EDGE_WIN,), jnp.int32),
            pltpu.VMEM((EDGE_WIN, d), jnp.float32),
            pltpu.VMEM((EDGE_WIN, d), jnp.float32),
            pltpu.VMEM((8, d), jnp.float32),
            pltpu.VMEM_SHARED((n_pad, d), jnp.float32),
            pltpu.SemaphoreType.DMA,
            pltpu.SemaphoreType.DMA,
            pltpu.SemaphoreType.DMA,
            pltpu.SemaphoreType.DMA,
            pltpu.SemaphoreType.DMA,
            pltpu.SemaphoreType.DMA,
        ],
    )
    return kern(proj0, proj1, row0, col0, val0, row1, col1, val1)


def _combine_body(masked_ref, part_ref, meta_ref, out_ref):
    meta = meta_ref[0, :]
    out_ref[...] = masked_ref[...] + STRENGTH * (
        part_ref[0] + part_ref[1] + meta[None, :])


def _combine(masked, part, meta_emb, block_rows=1000):
    n, d = masked.shape
    nb = n // block_rows
    return pl.pallas_call(
        _combine_body,
        grid=(nb,),
        in_specs=[
            pl.BlockSpec((block_rows, d), lambda b: (b, 0)),
            pl.BlockSpec((NC, block_rows, d), lambda b: (0, b, 0)),
            pl.BlockSpec((1, d), lambda b: (0, 0)),
        ],
        out_specs=pl.BlockSpec((block_rows, d), lambda b: (b, 0)),
        out_shape=jax.ShapeDtypeStruct((n, d), jnp.float32),
    )(masked, part, meta_emb)


def kernel(feat0, feat1, mp0_row, mp0_col, mp0_val, mp1_row, mp1_col, mp1_val,
           mask_idx0, mask_idx1, mask_token0, mask_token1,
           meta_emb0, meta_emb1, W0, W1):
    masked0, proj0 = _mask_project(feat0, mask_idx0, mask_token0, W0)
    masked1, proj1 = _mask_project(feat1, mask_idx1, mask_token1, W1)
    part0, part1 = _spmm_both(proj0, proj1, mp0_row, mp0_col, mp0_val,
                              mp1_row, mp1_col, mp1_val)
    out0 = _combine(masked0, part0, meta_emb0)
    out1 = _combine(masked1, part1, meta_emb1)
    return (out0, out1)


# speedup vs baseline: 10.8688x; 1.1951x over previous
import functools

import jax
import jax.numpy as jnp
from jax import lax
from jax.experimental import pallas as pl
from jax.experimental.pallas import tpu as pltpu
from jax.experimental.pallas import tpu_sc as plsc

STRENGTH = 0.1

NC = 2
NS = 16
LANES = 16

NBUF = 4

EDGE_WIN = 80


def _mask_project_body(feat_ref, midx_ref, tok_ref, w_ref, masked_ref, proj_ref,
                       *, block_rows):
    b = pl.program_id(0)
    feat = feat_ref[...]
    midx = midx_ref[0, :]
    rows = b * block_rows + lax.broadcasted_iota(
        jnp.int32, (block_rows, midx.shape[0]), 0)
    is_masked = jnp.any(rows == midx[None, :], axis=1)
    tok = tok_ref[0, :]
    masked = jnp.where(is_masked[:, None], tok[None, :], feat)
    masked_ref[...] = masked
    proj_ref[...] = lax.dot_general(
        masked, w_ref[...], (((1,), (1,)), ((), ())),
        preferred_element_type=jnp.float32,
        precision=lax.Precision.HIGHEST)


def _mask_project(feat, mask_idx, mask_token, w, block_rows=1000):
    n, d = feat.shape
    nb = n // block_rows
    nmask = mask_idx.shape[0]
    return pl.pallas_call(
        functools.partial(_mask_project_body, block_rows=block_rows),
        grid=(nb,),
        in_specs=[
            pl.BlockSpec((block_rows, d), lambda b: (b, 0)),
            pl.BlockSpec((1, nmask), lambda b: (0, 0)),
            pl.BlockSpec((1, d), lambda b: (0, 0)),
            pl.BlockSpec((d, d), lambda b: (0, 0)),
        ],
        out_specs=[
            pl.BlockSpec((block_rows, d), lambda b: (b, 0)),
            pl.BlockSpec((block_rows, d), lambda b: (b, 0)),
        ],
        out_shape=[
            jax.ShapeDtypeStruct((n, d), jnp.float32),
            jax.ShapeDtypeStruct((n, d), jnp.float32),
        ],
    )(feat, mask_idx.reshape(1, nmask), mask_token, w)


def _spmm_body(proj0, proj1, row0, col0, val0, row1, col1, val1,
               part0, part1,
               *refs, n_pad, d, e):
    ridx = refs[0:NBUF]
    cidx = refs[NBUF:2 * NBUF]
    vbuf = refs[2 * NBUF:3 * NBUF]
    sidx = refs[3 * NBUF:4 * NBUF]
    rows = refs[4 * NBUF:5 * NBUF]
    zbuf = refs[5 * NBUF]
    acc = refs[5 * NBUF + 1]
    semi = refs[5 * NBUF + 2:5 * NBUF + 2 + NBUF]
    semg = refs[5 * NBUF + 2 + NBUF:5 * NBUF + 2 + 2 * NBUF]
    sems = refs[5 * NBUF + 2 + 2 * NBUF:5 * NBUF + 2 + 3 * NBUF]

    c = lax.axis_index("c")
    s = lax.axis_index("s")
    wid = c * NS + s
    e_per_w = e // (NC * NS)
    n_win = e_per_w // EDGE_WIN
    rows_per_sub = n_pad // NS
    zrows = zbuf.shape[0]
    n_grp = EDGE_WIN // LANES
    n_peel = NBUF + 1

    def zfill(r, _):
        for j in range(d // LANES):
            zbuf[r, pl.ds(j * LANES, LANES)] = jnp.zeros((LANES,), jnp.float32)
        return 0
    lax.fori_loop(0, zrows, zfill, 0)

    def run_path(proj, row, col, val, part):
        base0 = wid * e_per_w

        def fire_idx(w, b):
            off = base0 + w * EDGE_WIN
            pltpu.async_copy(row.at[pl.ds(off, EDGE_WIN)], ridx[b], semi[b])
            pltpu.async_copy(col.at[pl.ds(off, EDGE_WIN)], cidx[b], semi[b])
            pltpu.async_copy(val.at[pl.ds(off, EDGE_WIN)], vbuf[b], semi[b])

        def wait_idx(w, b):
            off = base0 + w * EDGE_WIN
            pltpu.make_async_copy(row.at[pl.ds(off, EDGE_WIN)], ridx[b], semi[b]).wait()
            pltpu.make_async_copy(col.at[pl.ds(off, EDGE_WIN)], cidx[b], semi[b]).wait()
            pltpu.make_async_copy(val.at[pl.ds(off, EDGE_WIN)], vbuf[b], semi[b]).wait()

        def fire_gather(b):
            pltpu.async_copy(proj.at[cidx[b]], rows[b], semg[b])

        def wait_gather(b):
            pltpu.make_async_copy(proj.at[cidx[b]], rows[b], semg[b]).wait()

        def fire_scat(b):
            pltpu.async_copy(rows[b], acc.at[sidx[b]], sems[b], add=True)

        def wait_scat(b):
            pltpu.make_async_copy(rows[b], acc.at[sidx[b]], sems[b]).wait()

        def scale_and_stage(b):
            def grp(g, _):
                vv = vbuf[b][pl.ds(g * LANES, LANES)]
                sidx[b][pl.ds(g * LANES, LANES)] = ridx[b][pl.ds(g * LANES, LANES)]
                for t in range(LANES):
                    k = g * LANES + t
                    v = jnp.full((LANES,), vv[t], jnp.float32)
                    for j in range(d // LANES):
                        rows[b][k, pl.ds(j * LANES, LANES)] = (
                            rows[b][k, pl.ds(j * LANES, LANES)] * v)
                return 0
            lax.fori_loop(0, n_grp, grp, 0)

        def zero_step(t, _):
            pltpu.sync_copy(zbuf, acc.at[pl.ds(s * rows_per_sub + t * zrows, zrows)])
            return 0
        lax.fori_loop(0, rows_per_sub // zrows, zero_step, 0)
        plsc.subcore_barrier()

        def body(w, b, peeled):
            b2 = (b + 2) % NBUF
            if peeled:
                wait_idx(w + 2, b2)
                if w >= 2:
                    wait_scat(b2)
                fire_gather(b2)
            else:
                @pl.when(w + 2 < n_win)
                def _():
                    wait_idx(w + 2, b2)
                    wait_scat(b2)
                    fire_gather(b2)
            wait_gather(b)
            scale_and_stage(b)
            fire_scat(b)
            if peeled:
                if w + NBUF < n_win:
                    fire_idx(w + NBUF, b)
            else:
                @pl.when(w + NBUF < n_win)
                def _():
                    fire_idx(w + NBUF, b)

        for b in range(NBUF):
            fire_idx(b, b)
        wait_idx(0, 0)
        fire_gather(0)
        wait_idx(1, 1)
        fire_gather(1)
        for w in range(n_peel):
            body(w, w % NBUF, peeled=True)

        def quad(t, _):
            for q in range(NBUF):
                w = n_peel + NBUF * t + q
                body(w, (n_peel + q) % NBUF, peeled=False)
            return 0
        lax.fori_loop(0, (n_win - n_peel) // NBUF, quad, 0)

        for wlast in range(n_win - NBUF, n_win):
            wait_scat(wlast % NBUF)
        plsc.subcore_barrier()

        pltpu.sync_copy(acc.at[pl.ds(s * rows_per_sub, rows_per_sub)],
                        part.at[c, pl.ds(s * rows_per_sub, rows_per_sub)])
        plsc.subcore_barrier()

    run_path(proj0, row0, col0, val0, part0)
    run_path(proj1, row1, col1, val1, part1)


def _spmm_both(proj0, proj1, row0, col0, val0, row1, col1, val1):
    n, d = proj0.shape
    e = row0.shape[0]
    n_pad = ((n + 8 * NS - 1) // (8 * NS)) * (8 * NS)
    e_per_w = e // (NC * NS)
    n_win = e_per_w // EDGE_WIN
    n_peel = NBUF + 1
    assert e_per_w % EDGE_WIN == 0
    assert n_win > n_peel and (n_win - n_peel) % NBUF == 0
    mesh = plsc.VectorSubcoreMesh(core_axis_name="c", subcore_axis_name="s",
                                  num_cores=NC, num_subcores=NS)
    kern = pl.kernel(
        functools.partial(_spmm_body, n_pad=n_pad, d=d, e=e),
        out_type=[
            jax.ShapeDtypeStruct((NC, n_pad, d), jnp.float32),
            jax.ShapeDtypeStruct((NC, n_pad, d), jnp.float32),
        ],
        mesh=mesh,
        scratch_types=(
            [pltpu.VMEM((EDGE_WIN,), jnp.int32)] * NBUF
            + [pltpu.VMEM((EDGE_WIN,), jnp.int32)] * NBUF
            + [pltpu.VMEM((EDGE_WIN,), jnp.float32)] * NBUF
            + [pltpu.VMEM((EDGE_WIN,), jnp.int32)] * NBUF
            + [pltpu.VMEM((EDGE_WIN, d), jnp.float32)] * NBUF
            + [pltpu.VMEM((8, d), jnp.float32)]
            + [pltpu.VMEM_SHARED((n_pad, d), jnp.float32)]
            + [pltpu.SemaphoreType.DMA] * (3 * NBUF)
        ),
    )
    return kern(proj0, proj1, row0, col0, val0, row1, col1, val1)


def _combine_body(masked_ref, part_ref, meta_ref, out_ref):
    meta = meta_ref[0, :]
    out_ref[...] = masked_ref[...] + STRENGTH * (
        part_ref[0] + part_ref[1] + meta[None, :])


def _combine(masked, part, meta_emb, block_rows=1000):
    n, d = masked.shape
    nb = n // block_rows
    return pl.pallas_call(
        _combine_body,
        grid=(nb,),
        in_specs=[
            pl.BlockSpec((block_rows, d), lambda b: (b, 0)),
            pl.BlockSpec((NC, block_rows, d), lambda b: (0, b, 0)),
            pl.BlockSpec((1, d), lambda b: (0, 0)),
        ],
        out_specs=pl.BlockSpec((block_rows, d), lambda b: (b, 0)),
        out_shape=jax.ShapeDtypeStruct((n, d), jnp.float32),
    )(masked, part, meta_emb)


def kernel(feat0, feat1, mp0_row, mp0_col, mp0_val, mp1_row, mp1_col, mp1_val,
           mask_idx0, mask_idx1, mask_token0, mask_token1,
           meta_emb0, meta_emb1, W0, W1):
    masked0, proj0 = _mask_project(feat0, mask_idx0, mask_token0, W0)
    masked1, proj1 = _mask_project(feat1, mask_idx1, mask_token1, W1)
    part0, part1 = _spmm_both(proj0, proj1, mp0_row, mp0_col, mp0_val,
                              mp1_row, mp1_col, mp1_val)
    out0 = _combine(masked0, part0, meta_emb0)
    out1 = _combine(masked1, part1, meta_emb1)
    return (out0, out1)
